# Initial kernel scaffold; baseline (speedup 1.0000x reference)
import jax
import jax.numpy as jnp
from jax import lax
from jax.experimental import pallas as pl
from jax.experimental.pallas import tpu as pltpu
from jax.experimental.pallas import tpu_sc as plsc

_N = 10000
_D = 128
_HID = 256
_E = 320000
_NC = 2
_NS = 16
_NW = _NC * _NS
_CHUNK = 128
_EPT = 10112
_K = _EPT // _CHUNK
_EPAD = _EPT * _NW
_ACC_ROWS = 10240
_RPT = _ACC_ROWS // _NS
_OPT = _N // _NS


def _seg_builder(with_deg):
  mesh = plsc.VectorSubcoreMesh(core_axis_name="c", subcore_axis_name="s")
  out_type = [jax.ShapeDtypeStruct((_NC, _N, _D), jnp.float32)]
  scratch = [
      pltpu.VMEM((_K, _CHUNK), jnp.int32),
      pltpu.VMEM((_K, _CHUNK), jnp.int32),
      pltpu.VMEM((_CHUNK, _D), jnp.float32),
      pltpu.VMEM_SHARED((_ACC_ROWS, _D), jnp.float32),
      pltpu.SemaphoreType.DMA,
  ]
  if with_deg:
    out_type.append(jax.ShapeDtypeStruct((_NC, _N, 16), jnp.float32))
    scratch += [
        pltpu.VMEM((_CHUNK, 16), jnp.float32),
        pltpu.VMEM_SHARED((_ACC_ROWS, 16), jnp.float32),
    ]

  def body(*refs):
    if with_deg:
      (vals, src_h, dst_h, zr, zd, ones_h, out_h, deg_h,
       src_v, dst_v, rows_v, acc, sem, ones_v, dacc) = refs
    else:
      (vals, src_h, dst_h, zr, out_h,
       src_v, dst_v, rows_v, acc, sem) = refs
    c = lax.axis_index("c")
    s = lax.axis_index("s")
    w = s * _NC + c
    pltpu.sync_copy(zr, acc.at[pl.ds(s * _RPT, _RPT)])
    pltpu.sync_copy(src_h.at[w], src_v)
    pltpu.sync_copy(dst_h.at[w], dst_v)
    if with_deg:
      pltpu.sync_copy(zd, dacc.at[pl.ds(s * _RPT, _RPT)])
      pltpu.sync_copy(ones_h, ones_v)
    plsc.subcore_barrier()

    @pl.loop(0, _K)
    def _(j):
      pltpu.async_copy(vals.at[src_v.at[j]], rows_v, sem).wait()
      pltpu.sync_copy(rows_v, acc.at[dst_v.at[j]], add=True)
      if with_deg:
        pltpu.sync_copy(ones_v, dacc.at[dst_v.at[j]], add=True)

    plsc.subcore_barrier()
    pltpu.sync_copy(acc.at[pl.ds(s * _OPT, _OPT)],
                    out_h.at[c, pl.ds(s * _OPT, _OPT)])
    if with_deg:
      pltpu.sync_copy(dacc.at[pl.ds(s * _OPT, _OPT)],
                      deg_h.at[c, pl.ds(s * _OPT, _OPT)])

  return pl.kernel(body, out_type=out_type, mesh=mesh, scratch_types=scratch)


_seg_deg = _seg_builder(True)
_seg = _seg_builder(False)


def _dot_t(a, b):
  return lax.dot_general(a, b, (((1,), (1,)), ((), ())),
                         precision=lax.Precision.HIGHEST,
                         preferred_element_type=jnp.float32)


_BLK = 2000


def _dense1_body(s1p, degp, x, w1l, b1l, w1r, w2l, w2r, p2, q2):
  deg = degp[0][:, 0:1] + degp[1][:, 0:1]
  rinv = 1.0 / jnp.maximum(deg, 1.0)
  agg = (s1p[0] + s1p[1]) * rinv
  h = _dot_t(agg, w1l[...]) + b1l[...] + _dot_t(x[...], w1r[...])
  h = jnp.maximum(h, 0.0)
  p2[...] = _dot_t(h, w2l[...])
  q2[...] = _dot_t(h, w2r[...])


def _dense1(s1p, degp, x, W1l, b1l, W1r, W2l, W2r):
  return pl.pallas_call(
      _dense1_body,
      grid=(_N // _BLK,),
      in_specs=[
          pl.BlockSpec((2, _BLK, _D), lambda i: (0, i, 0)),
          pl.BlockSpec((2, _BLK, 16), lambda i: (0, i, 0)),
          pl.BlockSpec((_BLK, _D), lambda i: (i, 0)),
          pl.BlockSpec((_HID, _D), lambda i: (0, 0)),
          pl.BlockSpec((1, _HID), lambda i: (0, 0)),
          pl.BlockSpec((_HID, _D), lambda i: (0, 0)),
          pl.BlockSpec((_D, _HID), lambda i: (0, 0)),
          pl.BlockSpec((_D, _HID), lambda i: (0, 0)),
      ],
      out_specs=[
          pl.BlockSpec((_BLK, _D), lambda i: (i, 0)),
          pl.BlockSpec((_BLK, _D), lambda i: (i, 0)),
      ],
      out_shape=[jax.ShapeDtypeStruct((_N, _D), jnp.float32)] * 2,
  )(s1p, degp, x, W1l, b1l, W1r, W2l, W2r)


def _dense2_body(s2p, degp, q2, b2l, out):
  deg = degp[0][:, 0:1] + degp[1][:, 0:1]
  out[...] = (s2p[0] + s2p[1]) / jnp.maximum(deg, 1.0) + b2l[...] + q2[...]


def _dense2(s2p, degp, q2, b2l):
  return pl.pallas_call(
      _dense2_body,
      grid=(_N // _BLK,),
      in_specs=[
          pl.BlockSpec((2, _BLK, _D), lambda i: (0, i, 0)),
          pl.BlockSpec((2, _BLK, 16), lambda i: (0, i, 0)),
          pl.BlockSpec((_BLK, _D), lambda i: (i, 0)),
          pl.BlockSpec((1, _D), lambda i: (0, 0)),
      ],
      out_specs=pl.BlockSpec((_BLK, _D), lambda i: (i, 0)),
      out_shape=jax.ShapeDtypeStruct((_N, _D), jnp.float32),
  )(s2p, degp, q2, b2l)


def kernel(x, edge_index, W1l, b1l, W1r, W2l, b2l, W2r):
  src = edge_index[0].astype(jnp.int32)
  dst = edge_index[1].astype(jnp.int32)
  pad = _EPAD - _E
  i = jnp.arange(pad, dtype=jnp.int32)
  src_p = jnp.concatenate([src, (i * 37) % _N])
  dst_p = jnp.concatenate([dst, _N + (i % (_ACC_ROWS - _N))])
  src3 = src_p.reshape(_NW, _K, _CHUNK)
  dst3 = dst_p.reshape(_NW, _K, _CHUNK)
  zr = jnp.zeros((_RPT, _D), jnp.float32)
  zd = jnp.zeros((_RPT, 16), jnp.float32)
  ones = jnp.ones((_CHUNK, 16), jnp.float32)

  s1p, degp = _seg_deg(x, src3, dst3, zr, zd, ones)
  p2, q2 = _dense1(s1p, degp, x, W1l, b1l.reshape(1, -1), W1r, W2l, W2r)
  s2p = _seg(p2, src3, dst3, zr)
  return _dense2(s2p, degp, q2, b2l.reshape(1, -1))


# speedup vs baseline: 9.0806x; 9.0806x over previous
import jax
import jax.numpy as jnp
from jax import lax
from jax.experimental import pallas as pl
from jax.experimental.pallas import tpu as pltpu
from jax.experimental.pallas import tpu_sc as plsc

_N = 10000
_D = 128
_HID = 256
_E = 320000
_NC = 2
_NS = 16
_NW = _NC * _NS
_CHUNK = 128
_SUP = 8
_K = 80
_NSUP = _K // _SUP
_EPT = _K * _CHUNK
_EPAD = _EPT * _NW
_ACC_ROWS = 10112
_RPT = _ACC_ROWS // _NS
_OPT = 624
_TAIL = _N - _OPT * _NS
_DEG_N = 10240
_DPT = _DEG_N // _NS


def _chunks(total):
  out, o = [], 0
  while o < total:
    n = min(_CHUNK, total - o)
    out.append((o, n))
    o += n
  return out


def _seg_builder(with_deg):
  mesh = plsc.VectorSubcoreMesh(core_axis_name="c", subcore_axis_name="s")
  out_type = [jax.ShapeDtypeStruct((_NC, _N, _D), jnp.float32)]
  scratch = [
      pltpu.VMEM((_SUP, _CHUNK), jnp.int32),
      pltpu.VMEM((_SUP, _CHUNK), jnp.int32),
      pltpu.VMEM((_CHUNK, _D), jnp.float32),
      pltpu.VMEM_SHARED((_ACC_ROWS, _D), jnp.float32),
      pltpu.SemaphoreType.DMA,
  ]
  if with_deg:
    out_type.append(jax.ShapeDtypeStruct((_NC, _DEG_N), jnp.float32))
    scratch += [
        pltpu.VMEM((_CHUNK,), jnp.float32),
        pltpu.VMEM((_DPT,), jnp.float32),
        pltpu.VMEM_SHARED((_DEG_N,), jnp.float32),
    ]

  def body(*refs):
    deg_h = ones_v = dstage = dacc = None
    if with_deg:
      (vals, src_h, dst_h, zr, out_h, deg_h,
       src_v, dst_v, rows_v, acc, sem, ones_v, dstage, dacc) = refs
    else:
      (vals, src_h, dst_h, zr, out_h,
       src_v, dst_v, rows_v, acc, sem) = refs
    c = lax.axis_index("c")
    s = lax.axis_index("s")
    w = s * _NC + c
    pltpu.sync_copy(zr, rows_v)
    if with_deg:
      for p in range(_CHUNK // 16):
        ones_v[pl.ds(p * 16, 16)] = jnp.ones((16,), jnp.float32)
      for p in range(_DPT // 16):
        dstage[pl.ds(p * 16, 16)] = jnp.zeros((16,), jnp.float32)
      pltpu.sync_copy(dstage, dacc.at[pl.ds(s * _DPT, _DPT)])
    for o, n in _chunks(_RPT):
      pltpu.sync_copy(rows_v.at[pl.ds(0, n)],
                      acc.at[pl.ds(s * _RPT + o, n)])
    plsc.subcore_barrier()

    @pl.loop(0, _NSUP)
    def _(g):
      pltpu.sync_copy(src_h.at[w, pl.ds(g * _SUP, _SUP)], src_v)
      pltpu.sync_copy(dst_h.at[w, pl.ds(g * _SUP, _SUP)], dst_v)
      for j in range(_SUP):
        pltpu.async_copy(vals.at[src_v.at[j]], rows_v, sem).wait()
        pltpu.sync_copy(rows_v, acc.at[dst_v.at[j]], add=True)
        if with_deg:
          pltpu.sync_copy(ones_v, dacc.at[dst_v.at[j]], add=True)

    plsc.subcore_barrier()

    def copy_out(base, total):
      for o, n in _chunks(total):
        pltpu.sync_copy(acc.at[pl.ds(base + o, n)], rows_v.at[pl.ds(0, n)])
        pltpu.sync_copy(rows_v.at[pl.ds(0, n)],
                        out_h.at[c, pl.ds(base + o, n)])

    copy_out(s * _OPT, _OPT)
    if with_deg:
      pltpu.sync_copy(dacc.at[pl.ds(s * _DPT, _DPT)], dstage)
      pltpu.sync_copy(dstage, deg_h.at[c, pl.ds(s * _DPT, _DPT)])

    @pl.when(s == _NS - 1)
    def _():
      copy_out(_OPT * _NS, _TAIL)

  return pl.kernel(body, out_type=out_type, mesh=mesh, scratch_types=scratch)


_seg_deg = _seg_builder(True)
_seg = _seg_builder(False)


def _dot_t(a, b):
  return lax.dot_general(a, b, (((1,), (1,)), ((), ())),
                         precision=lax.Precision.HIGHEST,
                         preferred_element_type=jnp.float32)


_BLK = 2000


def _dense1_body(s1p, degp, x, w1l, b1l, w1r, w2l, w2r, p2, q2):
  deg = degp[0] + degp[1]
  rinv = 1.0 / jnp.maximum(deg, 1.0)
  agg = (s1p[0] + s1p[1]) * rinv
  h = _dot_t(agg, w1l[...]) + b1l[...] + _dot_t(x[...], w1r[...])
  h = jnp.maximum(h, 0.0)
  p2[...] = _dot_t(h, w2l[...])
  q2[...] = _dot_t(h, w2r[...])


def _dense1(s1p, degp, x, W1l, b1l, W1r, W2l, W2r):
  return pl.pallas_call(
      _dense1_body,
      grid=(_N // _BLK,),
      in_specs=[
          pl.BlockSpec((2, _BLK, _D), lambda i: (0, i, 0)),
          pl.BlockSpec((2, _BLK, 1), lambda i: (0, i, 0)),
          pl.BlockSpec((_BLK, _D), lambda i: (i, 0)),
          pl.BlockSpec((_HID, _D), lambda i: (0, 0)),
          pl.BlockSpec((1, _HID), lambda i: (0, 0)),
          pl.BlockSpec((_HID, _D), lambda i: (0, 0)),
          pl.BlockSpec((_D, _HID), lambda i: (0, 0)),
          pl.BlockSpec((_D, _HID), lambda i: (0, 0)),
      ],
      out_specs=[
          pl.BlockSpec((_BLK, _D), lambda i: (i, 0)),
          pl.BlockSpec((_BLK, _D), lambda i: (i, 0)),
      ],
      out_shape=[jax.ShapeDtypeStruct((_N, _D), jnp.float32)] * 2,
  )(s1p, degp, x, W1l, b1l, W1r, W2l, W2r)


def _dense2_body(s2p, degp, q2, b2l, out):
  deg = degp[0] + degp[1]
  out[...] = (s2p[0] + s2p[1]) / jnp.maximum(deg, 1.0) + b2l[...] + q2[...]


def _dense2(s2p, degp, q2, b2l):
  return pl.pallas_call(
      _dense2_body,
      grid=(_N // _BLK,),
      in_specs=[
          pl.BlockSpec((2, _BLK, _D), lambda i: (0, i, 0)),
          pl.BlockSpec((2, _BLK, 1), lambda i: (0, i, 0)),
          pl.BlockSpec((_BLK, _D), lambda i: (i, 0)),
          pl.BlockSpec((1, _D), lambda i: (0, 0)),
      ],
      out_specs=pl.BlockSpec((_BLK, _D), lambda i: (i, 0)),
      out_shape=jax.ShapeDtypeStruct((_N, _D), jnp.float32),
  )(s2p, degp, q2, b2l)


def kernel(x, edge_index, W1l, b1l, W1r, W2l, b2l, W2r):
  src = edge_index[0].astype(jnp.int32)
  dst = edge_index[1].astype(jnp.int32)
  pad = _EPAD - _E
  i = jnp.arange(pad, dtype=jnp.int32)
  src_p = jnp.concatenate([src, (i * 37) % _N])
  dst_p = jnp.concatenate([dst, _N + (i % (_ACC_ROWS - _N))])
  src3 = src_p.reshape(_NW, _K, _CHUNK)
  dst3 = dst_p.reshape(_NW, _K, _CHUNK)
  zr = jnp.zeros((_CHUNK, _D), jnp.float32)

  s1p, deg_flat = _seg_deg(x, src3, dst3, zr)
  degp = deg_flat[:, :_N, None]
  p2, q2 = _dense1(s1p, degp, x, W1l, b1l.reshape(1, -1), W1r, W2l, W2r)
  (s2p,) = _seg(p2, src3, dst3, zr)
  return _dense2(s2p, degp, q2, b2l.reshape(1, -1))


# speedup vs baseline: 11.8136x; 1.3010x over previous
import jax
import jax.numpy as jnp
from jax import lax
from jax.experimental import pallas as pl
from jax.experimental.pallas import tpu as pltpu
from jax.experimental.pallas import tpu_sc as plsc

_N = 10000
_D = 128
_HID = 256
_E = 320000
_NC = 2
_NS = 16
_NW = _NC * _NS
_CHUNK = 128
_SUP = 16
_K = 80
_NSUP = _K // _SUP
_EPT = _K * _CHUNK
_EPAD = _EPT * _NW
_ACC_ROWS = 10112
_RPT = _ACC_ROWS // _NS
_OPT = 624
_TAIL = _N - _OPT * _NS
_DEG_N = 10240
_DPT = _DEG_N // _NS


def _chunks(total):
  out, o = [], 0
  while o < total:
    n = min(_CHUNK, total - o)
    out.append((o, n))
    o += n
  return out


def _seg_builder(with_deg):
  mesh = plsc.VectorSubcoreMesh(core_axis_name="c", subcore_axis_name="s")
  out_type = [jax.ShapeDtypeStruct((_NC, _N, _D), jnp.float32)]
  scratch = [
      pltpu.VMEM((_SUP, _CHUNK), jnp.int32),
      pltpu.VMEM((_SUP, _CHUNK), jnp.int32),
      pltpu.VMEM((_SUP, _CHUNK), jnp.int32),
      pltpu.VMEM((_SUP, _CHUNK), jnp.int32),
      pltpu.VMEM((_CHUNK, _D), jnp.float32),
      pltpu.VMEM((_CHUNK, _D), jnp.float32),
      pltpu.VMEM_SHARED((_ACC_ROWS, _D), jnp.float32),
      pltpu.SemaphoreType.DMA,
      pltpu.SemaphoreType.DMA,
      pltpu.SemaphoreType.DMA,
  ]
  if with_deg:
    out_type.append(jax.ShapeDtypeStruct((_NC, _DEG_N), jnp.float32))
    scratch += [
        pltpu.VMEM((_CHUNK,), jnp.float32),
        pltpu.VMEM((_DPT,), jnp.float32),
        pltpu.VMEM_SHARED((_DEG_N,), jnp.float32),
        pltpu.SemaphoreType.DMA,
    ]

  def body(*refs):
    deg_h = ones_v = dstage = dacc = sem_d = None
    if with_deg:
      (vals, src_h, dst_h, zr, out_h, deg_h,
       src_a, src_b, dst_a, dst_b, rows_a, rows_b, acc,
       sem_g, sem_s, sem_i, ones_v, dstage, dacc, sem_d) = refs
    else:
      (vals, src_h, dst_h, zr, out_h,
       src_a, src_b, dst_a, dst_b, rows_a, rows_b, acc,
       sem_g, sem_s, sem_i) = refs
    src_v = (src_a, src_b)
    dst_v = (dst_a, dst_b)
    rows_v = rows_a
    rows = (rows_a, rows_b)
    c = lax.axis_index("c")
    s = lax.axis_index("s")
    w = s * _NC + c
    pltpu.sync_copy(zr, rows_v)
    if with_deg:
      for p in range(_CHUNK // 16):
        ones_v[pl.ds(p * 16, 16)] = jnp.ones((16,), jnp.float32)
      for p in range(_DPT // 16):
        dstage[pl.ds(p * 16, 16)] = jnp.zeros((16,), jnp.float32)
      pltpu.sync_copy(dstage, dacc.at[pl.ds(s * _DPT, _DPT)])
    for o, n in _chunks(_RPT):
      pltpu.sync_copy(rows_v.at[pl.ds(0, n)],
                      acc.at[pl.ds(s * _RPT + o, n)])
    plsc.subcore_barrier()

    pltpu.sync_copy(src_h.at[w, pl.ds(0, _SUP)], src_v[0])
    pltpu.sync_copy(dst_h.at[w, pl.ds(0, _SUP)], dst_v[0])
    g_h, s_h, d_h, i_h = {}, {}, {}, {}
    g_h[0] = pltpu.async_copy(vals.at[src_v[0].at[0]], rows[0], sem_g)
    for k in range(_K):
      g, j = divmod(k, _SUP)
      b = k % 2
      g_h[k].wait()
      if k >= 1:
        s_h[k - 1].wait()
        if with_deg:
          d_h[k - 1].wait()
      if j == 0 and g + 1 < _NSUP:
        i_h[g + 1] = (
            pltpu.async_copy(src_h.at[w, pl.ds((g + 1) * _SUP, _SUP)],
                             src_v[(g + 1) % 2], sem_i),
            pltpu.async_copy(dst_h.at[w, pl.ds((g + 1) * _SUP, _SUP)],
                             dst_v[(g + 1) % 2], sem_i))
      if k + 1 < _K:
        gn, jn = divmod(k + 1, _SUP)
        if jn == 0:
          i_h[gn][0].wait()
          i_h[gn][1].wait()
        g_h[k + 1] = pltpu.async_copy(vals.at[src_v[gn % 2].at[jn]],
                                      rows[1 - b], sem_g)
      s_h[k] = pltpu.async_copy(rows[b], acc.at[dst_v[g % 2].at[j]],
                                sem_s, add=True)
      if with_deg:
        d_h[k] = pltpu.async_copy(ones_v, dacc.at[dst_v[g % 2].at[j]],
                                  sem_d, add=True)
    s_h[_K - 1].wait()
    if with_deg:
      d_h[_K - 1].wait()

    plsc.subcore_barrier()

    def copy_out(base, total):
      for o, n in _chunks(total):
        pltpu.sync_copy(acc.at[pl.ds(base + o, n)], rows_v.at[pl.ds(0, n)])
        pltpu.sync_copy(rows_v.at[pl.ds(0, n)],
                        out_h.at[c, pl.ds(base + o, n)])

    copy_out(s * _OPT, _OPT)
    if with_deg:
      pltpu.sync_copy(dacc.at[pl.ds(s * _DPT, _DPT)], dstage)
      pltpu.sync_copy(dstage, deg_h.at[c, pl.ds(s * _DPT, _DPT)])

    @pl.when(s == _NS - 1)
    def _():
      copy_out(_OPT * _NS, _TAIL)

  return pl.kernel(body, out_type=out_type, mesh=mesh, scratch_types=scratch)


_seg_deg = _seg_builder(True)
_seg = _seg_builder(False)


def _dot_t(a, b):
  return lax.dot_general(a, b, (((1,), (1,)), ((), ())),
                         precision=lax.Precision.HIGHEST,
                         preferred_element_type=jnp.float32)


_BLK = 2000


def _dense1_body(s1p, degp, x, w1l, b1l, w1r, w2l, w2r, p2, q2):
  deg = degp[0] + degp[1]
  rinv = 1.0 / jnp.maximum(deg, 1.0)
  agg = (s1p[0] + s1p[1]) * rinv
  h = _dot_t(agg, w1l[...]) + b1l[...] + _dot_t(x[...], w1r[...])
  h = jnp.maximum(h, 0.0)
  p2[...] = _dot_t(h, w2l[...])
  q2[...] = _dot_t(h, w2r[...])


def _dense1(s1p, degp, x, W1l, b1l, W1r, W2l, W2r):
  return pl.pallas_call(
      _dense1_body,
      grid=(_N // _BLK,),
      in_specs=[
          pl.BlockSpec((2, _BLK, _D), lambda i: (0, i, 0)),
          pl.BlockSpec((2, _BLK, 1), lambda i: (0, i, 0)),
          pl.BlockSpec((_BLK, _D), lambda i: (i, 0)),
          pl.BlockSpec((_HID, _D), lambda i: (0, 0)),
          pl.BlockSpec((1, _HID), lambda i: (0, 0)),
          pl.BlockSpec((_HID, _D), lambda i: (0, 0)),
          pl.BlockSpec((_D, _HID), lambda i: (0, 0)),
          pl.BlockSpec((_D, _HID), lambda i: (0, 0)),
      ],
      out_specs=[
          pl.BlockSpec((_BLK, _D), lambda i: (i, 0)),
          pl.BlockSpec((_BLK, _D), lambda i: (i, 0)),
      ],
      out_shape=[jax.ShapeDtypeStruct((_N, _D), jnp.float32)] * 2,
  )(s1p, degp, x, W1l, b1l, W1r, W2l, W2r)


def _dense2_body(s2p, degp, q2, b2l, out):
  deg = degp[0] + degp[1]
  out[...] = (s2p[0] + s2p[1]) / jnp.maximum(deg, 1.0) + b2l[...] + q2[...]


def _dense2(s2p, degp, q2, b2l):
  return pl.pallas_call(
      _dense2_body,
      grid=(_N // _BLK,),
      in_specs=[
          pl.BlockSpec((2, _BLK, _D), lambda i: (0, i, 0)),
          pl.BlockSpec((2, _BLK, 1), lambda i: (0, i, 0)),
          pl.BlockSpec((_BLK, _D), lambda i: (i, 0)),
          pl.BlockSpec((1, _D), lambda i: (0, 0)),
      ],
      out_specs=pl.BlockSpec((_BLK, _D), lambda i: (i, 0)),
      out_shape=jax.ShapeDtypeStruct((_N, _D), jnp.float32),
  )(s2p, degp, q2, b2l)


def kernel(x, edge_index, W1l, b1l, W1r, W2l, b2l, W2r):
  src = edge_index[0].astype(jnp.int32)
  dst = edge_index[1].astype(jnp.int32)
  pad = _EPAD - _E
  i = jnp.arange(pad, dtype=jnp.int32)
  src_p = jnp.concatenate([src, (i * 37) % _N])
  dst_p = jnp.concatenate([dst, _N + (i % (_ACC_ROWS - _N))])
  src3 = src_p.reshape(_NW, _K, _CHUNK)
  dst3 = dst_p.reshape(_NW, _K, _CHUNK)
  zr = jnp.zeros((_CHUNK, _D), jnp.float32)

  s1p, deg_flat = _seg_deg(x, src3, dst3, zr)
  degp = deg_flat[:, :_N, None]
  p2, q2 = _dense1(s1p, degp, x, W1l, b1l.reshape(1, -1), W1r, W2l, W2r)
  (s2p,) = _seg(p2, src3, dst3, zr)
  return _dense2(s2p, degp, q2, b2l.reshape(1, -1))


# speedup vs baseline: 13.5539x; 1.1473x over previous
import jax
import jax.numpy as jnp
import numpy as np
from jax import lax
from jax.experimental import pallas as pl
from jax.experimental.pallas import tpu as pltpu
from jax.experimental.pallas import tpu_sc as plsc

_N = 10000
_D = 128
_HID = 256
_E = 320000
_NC = 2
_NS = 16
_NW = _NC * _NS
_CHUNK = 128
_SUP = 16
_K = 80
_NSUP = _K // _SUP
_EPT = _K * _CHUNK
_EPAD = _EPT * _NW
_ACC_ROWS = 10112
_RPT = _ACC_ROWS // _NS
_OPT = 624
_TAIL = _N - _OPT * _NS
_DEG_N = 10240
_DPT = _DEG_N // _NS


def _chunks(total):
  out, o = [], 0
  while o < total:
    n = min(_CHUNK, total - o)
    out.append((o, n))
    o += n
  return out


def _seg_builder(with_deg):
  mesh = plsc.VectorSubcoreMesh(core_axis_name="c", subcore_axis_name="s")
  out_type = [jax.ShapeDtypeStruct((_NC, _N, _D), jnp.float32)]
  scratch = [
      pltpu.VMEM((_SUP, _CHUNK), jnp.int32),
      pltpu.VMEM((_SUP, _CHUNK), jnp.int32),
      pltpu.VMEM((_SUP, _CHUNK), jnp.int32),
      pltpu.VMEM((_SUP, _CHUNK), jnp.int32),
      pltpu.VMEM((_CHUNK, _D), jnp.float32),
      pltpu.VMEM((_CHUNK, _D), jnp.float32),
      pltpu.VMEM_SHARED((_ACC_ROWS, _D), jnp.float32),
      pltpu.SemaphoreType.DMA,
      pltpu.SemaphoreType.DMA,
      pltpu.SemaphoreType.DMA,
  ]
  if with_deg:
    out_type.append(jax.ShapeDtypeStruct((_NC, _DEG_N), jnp.float32))
    scratch += [
        pltpu.VMEM((_CHUNK,), jnp.float32),
        pltpu.VMEM((_DPT,), jnp.float32),
        pltpu.VMEM_SHARED((_DEG_N,), jnp.float32),
        pltpu.SemaphoreType.DMA,
    ]

  def body(*refs):
    deg_h = ones_v = dstage = dacc = sem_d = None
    if with_deg:
      (vals, src_h, dst_h, zr, out_h, deg_h,
       src_a, src_b, dst_a, dst_b, rows_a, rows_b, acc,
       sem_g, sem_s, sem_i, ones_v, dstage, dacc, sem_d) = refs
    else:
      (vals, src_h, dst_h, zr, out_h,
       src_a, src_b, dst_a, dst_b, rows_a, rows_b, acc,
       sem_g, sem_s, sem_i) = refs
    src_v = (src_a, src_b)
    dst_v = (dst_a, dst_b)
    rows_v = rows_a
    rows = (rows_a, rows_b)
    c = lax.axis_index("c")
    s = lax.axis_index("s")
    w = s * _NC + c
    pltpu.sync_copy(zr, rows_v)
    if with_deg:
      for p in range(_CHUNK // 16):
        ones_v[pl.ds(p * 16, 16)] = jnp.ones((16,), jnp.float32)
      for p in range(_DPT // 16):
        dstage[pl.ds(p * 16, 16)] = jnp.zeros((16,), jnp.float32)
      pltpu.sync_copy(dstage, dacc.at[pl.ds(s * _DPT, _DPT)])
    for o, n in _chunks(_RPT):
      pltpu.sync_copy(rows_v.at[pl.ds(0, n)],
                      acc.at[pl.ds(s * _RPT + o, n)])
    plsc.subcore_barrier()

    pltpu.sync_copy(src_h.at[w, pl.ds(0, _SUP)], src_v[0])
    pltpu.sync_copy(dst_h.at[w, pl.ds(0, _SUP)], dst_v[0])
    g_h, s_h, d_h, i_h = {}, {}, {}, {}
    g_h[0] = pltpu.async_copy(vals.at[src_v[0].at[0]], rows[0], sem_g)
    for k in range(_K):
      g, j = divmod(k, _SUP)
      b = k % 2
      g_h[k].wait()
      if k >= 1:
        s_h[k - 1].wait()
        if with_deg:
          d_h[k - 1].wait()
      if j == 0 and g + 1 < _NSUP:
        i_h[g + 1] = (
            pltpu.async_copy(src_h.at[w, pl.ds((g + 1) * _SUP, _SUP)],
                             src_v[(g + 1) % 2], sem_i),
            pltpu.async_copy(dst_h.at[w, pl.ds((g + 1) * _SUP, _SUP)],
                             dst_v[(g + 1) % 2], sem_i))
      if k + 1 < _K:
        gn, jn = divmod(k + 1, _SUP)
        if jn == 0:
          i_h[gn][0].wait()
          i_h[gn][1].wait()
        g_h[k + 1] = pltpu.async_copy(vals.at[src_v[gn % 2].at[jn]],
                                      rows[1 - b], sem_g)
      s_h[k] = pltpu.async_copy(rows[b], acc.at[dst_v[g % 2].at[j]],
                                sem_s, add=True)
      if with_deg:
        d_h[k] = pltpu.async_copy(ones_v, dacc.at[dst_v[g % 2].at[j]],
                                  sem_d, add=True)
    s_h[_K - 1].wait()
    if with_deg:
      d_h[_K - 1].wait()

    plsc.subcore_barrier()

    def copy_out(base, total):
      for o, n in _chunks(total):
        pltpu.sync_copy(acc.at[pl.ds(base + o, n)], rows_v.at[pl.ds(0, n)])
        pltpu.sync_copy(rows_v.at[pl.ds(0, n)],
                        out_h.at[c, pl.ds(base + o, n)])

    copy_out(s * _OPT, _OPT)
    if with_deg:
      pltpu.sync_copy(dacc.at[pl.ds(s * _DPT, _DPT)], dstage)
      pltpu.sync_copy(dstage, deg_h.at[c, pl.ds(s * _DPT, _DPT)])

    @pl.when(s == _NS - 1)
    def _():
      copy_out(_OPT * _NS, _TAIL)

  return pl.kernel(body, out_type=out_type, mesh=mesh, scratch_types=scratch)


_seg_deg = _seg_builder(True)
_seg = _seg_builder(False)


def _dot_t(a, b):
  return lax.dot_general(a, b, (((1,), (1,)), ((), ())),
                         precision=lax.Precision.DEFAULT,
                         preferred_element_type=jnp.float32)


_BLK = 2000


def _dense0_body(x, w1r, xr):
  xr[...] = _dot_t(x[...], w1r[...])


def _dense0(x, W1r):
  return pl.pallas_call(
      _dense0_body,
      grid=(_N // _BLK,),
      in_specs=[
          pl.BlockSpec((_BLK, _D), lambda i: (i, 0)),
          pl.BlockSpec((_HID, _D), lambda i: (0, 0)),
      ],
      out_specs=pl.BlockSpec((_BLK, _HID), lambda i: (i, 0)),
      out_shape=jax.ShapeDtypeStruct((_N, _HID), jnp.float32),
  )(x, W1r)


def _dense1_body(s1p, degp, xr, w1l, b1l, w2l, w2r, p2, q2):
  deg = degp[0] + degp[1]
  rinv = 1.0 / jnp.maximum(deg, 1.0)
  agg = (s1p[0] + s1p[1]) * rinv
  h = _dot_t(agg, w1l[...]) + b1l[...] + xr[...]
  h = jnp.maximum(h, 0.0)
  p2[...] = _dot_t(h, w2l[...])
  q2[...] = _dot_t(h, w2r[...])


def _dense1(s1p, degp, xr, W1l, b1l, W2l, W2r):
  return pl.pallas_call(
      _dense1_body,
      grid=(_N // _BLK,),
      in_specs=[
          pl.BlockSpec((2, _BLK, _D), lambda i: (0, i, 0)),
          pl.BlockSpec((2, _BLK, 1), lambda i: (0, i, 0)),
          pl.BlockSpec((_BLK, _HID), lambda i: (i, 0)),
          pl.BlockSpec((_HID, _D), lambda i: (0, 0)),
          pl.BlockSpec((1, _HID), lambda i: (0, 0)),
          pl.BlockSpec((_D, _HID), lambda i: (0, 0)),
          pl.BlockSpec((_D, _HID), lambda i: (0, 0)),
      ],
      out_specs=[
          pl.BlockSpec((_BLK, _D), lambda i: (i, 0)),
          pl.BlockSpec((_BLK, _D), lambda i: (i, 0)),
      ],
      out_shape=[jax.ShapeDtypeStruct((_N, _D), jnp.float32)] * 2,
  )(s1p, degp, xr, W1l, b1l, W2l, W2r)


def _dense2_body(s2p, degp, q2, b2l, out):
  deg = degp[0] + degp[1]
  out[...] = (s2p[0] + s2p[1]) / jnp.maximum(deg, 1.0) + b2l[...] + q2[...]


def _dense2(s2p, degp, q2, b2l):
  return pl.pallas_call(
      _dense2_body,
      grid=(_N // _BLK,),
      in_specs=[
          pl.BlockSpec((2, _BLK, _D), lambda i: (0, i, 0)),
          pl.BlockSpec((2, _BLK, 1), lambda i: (0, i, 0)),
          pl.BlockSpec((_BLK, _D), lambda i: (i, 0)),
          pl.BlockSpec((1, _D), lambda i: (0, 0)),
      ],
      out_specs=pl.BlockSpec((_BLK, _D), lambda i: (i, 0)),
      out_shape=jax.ShapeDtypeStruct((_N, _D), jnp.float32),
  )(s2p, degp, q2, b2l)


def kernel(x, edge_index, W1l, b1l, W1r, W2l, b2l, W2r):
  src = edge_index[0].astype(jnp.int32)
  dst = edge_index[1].astype(jnp.int32)
  pad = _EPAD - _E
  i = np.arange(pad)
  pad_src = jnp.asarray((i * 37) % _N, dtype=jnp.int32)
  pad_dst = jnp.asarray(_N + (i % (_ACC_ROWS - _N)), dtype=jnp.int32)
  src3 = jnp.concatenate([src, pad_src]).reshape(_NW, _K, _CHUNK)
  dst3 = jnp.concatenate([dst, pad_dst]).reshape(_NW, _K, _CHUNK)
  zr = jnp.zeros((_CHUNK, _D), jnp.float32)

  xr = _dense0(x, W1r)
  s1p, deg_flat = _seg_deg(x, src3, dst3, zr)
  degp = deg_flat[:, :_N, None]
  p2, q2 = _dense1(s1p, degp, xr, W1l, b1l.reshape(1, -1), W2l, W2r)
  (s2p,) = _seg(p2, src3, dst3, zr)
  return _dense2(s2p, degp, q2, b2l.reshape(1, -1))


# speedup vs baseline: 14.1980x; 1.0475x over previous
import jax
import jax.numpy as jnp
import numpy as np
from jax import lax
from jax.experimental import pallas as pl
from jax.experimental.pallas import tpu as pltpu
from jax.experimental.pallas import tpu_sc as plsc

_N = 10000
_D = 128
_HID = 256
_E = 320000
_NC = 2
_NS = 16
_NW = _NC * _NS
_CHUNK = 128
_SUP = 16
_K = 80
_NSUP = _K // _SUP
_EPT = _K * _CHUNK
_EPAD = _EPT * _NW
_ACC_ROWS = 10112
_RPT = _ACC_ROWS // _NS
_OPT = 624
_TAIL = _N - _OPT * _NS
_DEG_N = 10240
_DPT = _DEG_N // _NS


def _chunks(total):
  out, o = [], 0
  while o < total:
    n = min(_CHUNK, total - o)
    out.append((o, n))
    o += n
  return out


def _seg_builder(with_deg):
  mesh = plsc.VectorSubcoreMesh(core_axis_name="c", subcore_axis_name="s")
  out_type = [jax.ShapeDtypeStruct((_NC, _N, _D), jnp.float32)]
  scratch = [
      pltpu.VMEM((_SUP, _CHUNK), jnp.int32),
      pltpu.VMEM((_SUP, _CHUNK), jnp.int32),
      pltpu.VMEM((_SUP, _CHUNK), jnp.int32),
      pltpu.VMEM((_SUP, _CHUNK), jnp.int32),
      pltpu.VMEM((_CHUNK, _D), jnp.float32),
      pltpu.VMEM((_CHUNK, _D), jnp.float32),
      pltpu.VMEM_SHARED((_ACC_ROWS, _D), jnp.float32),
      pltpu.SemaphoreType.DMA,
      pltpu.SemaphoreType.DMA,
      pltpu.SemaphoreType.DMA,
  ]
  if with_deg:
    out_type.append(jax.ShapeDtypeStruct((_NC, _DEG_N), jnp.float32))
    scratch += [
        pltpu.VMEM((_CHUNK,), jnp.float32),
        pltpu.VMEM((_DPT,), jnp.float32),
        pltpu.VMEM_SHARED((_DEG_N,), jnp.float32),
        pltpu.SemaphoreType.DMA,
    ]

  def body(*refs):
    deg_h = ones_v = dstage = dacc = sem_d = None
    if with_deg:
      (vals, edge_h, zr, out_h, deg_h,
       src_a, src_b, dst_a, dst_b, rows_a, rows_b, acc,
       sem_g, sem_s, sem_i, ones_v, dstage, dacc, sem_d) = refs
    else:
      (vals, edge_h, zr, out_h,
       src_a, src_b, dst_a, dst_b, rows_a, rows_b, acc,
       sem_g, sem_s, sem_i) = refs
    src_v = (src_a, src_b)
    dst_v = (dst_a, dst_b)
    rows_v = rows_a
    rows = (rows_b, rows_a)
    c = lax.axis_index("c")
    s = lax.axis_index("s")
    w = s * _NC + c
    i0 = (pltpu.async_copy(edge_h.at[0, w, pl.ds(0, _SUP)], src_v[0], sem_i),
          pltpu.async_copy(edge_h.at[1, w, pl.ds(0, _SUP)], dst_v[0], sem_i))
    pltpu.sync_copy(zr, rows_v)
    z_h = []
    for o, n in _chunks(_RPT):
      z_h.append(pltpu.async_copy(rows_v.at[pl.ds(0, n)],
                                  acc.at[pl.ds(s * _RPT + o, n)], sem_s))
    if with_deg:
      for p in range(_CHUNK // 16):
        ones_v[pl.ds(p * 16, 16)] = jnp.ones((16,), jnp.float32)
      for p in range(_DPT // 16):
        dstage[pl.ds(p * 16, 16)] = jnp.zeros((16,), jnp.float32)
      pltpu.sync_copy(dstage, dacc.at[pl.ds(s * _DPT, _DPT)])
    i0[0].wait()
    g0 = pltpu.async_copy(vals.at[src_v[0].at[0]], rows[0], sem_g)
    for h in z_h:
      h.wait()
    plsc.subcore_barrier()

    i0[1].wait()
    g_h, s_h, d_h, i_h = {}, {}, {}, {}
    g_h[0] = g0
    for k in range(_K):
      g, j = divmod(k, _SUP)
      b = k % 2
      g_h[k].wait()
      if k >= 1:
        s_h[k - 1].wait()
        if with_deg:
          d_h[k - 1].wait()
      if j == 0 and g + 1 < _NSUP:
        i_h[g + 1] = (
            pltpu.async_copy(edge_h.at[0, w, pl.ds((g + 1) * _SUP, _SUP)],
                             src_v[(g + 1) % 2], sem_i),
            pltpu.async_copy(edge_h.at[1, w, pl.ds((g + 1) * _SUP, _SUP)],
                             dst_v[(g + 1) % 2], sem_i))
      if k + 1 < _K:
        gn, jn = divmod(k + 1, _SUP)
        if jn == 0:
          i_h[gn][0].wait()
          i_h[gn][1].wait()
        g_h[k + 1] = pltpu.async_copy(vals.at[src_v[gn % 2].at[jn]],
                                      rows[1 - b], sem_g)
      s_h[k] = pltpu.async_copy(rows[b], acc.at[dst_v[g % 2].at[j]],
                                sem_s, add=True)
      if with_deg:
        d_h[k] = pltpu.async_copy(ones_v, dacc.at[dst_v[g % 2].at[j]],
                                  sem_d, add=True)
    s_h[_K - 1].wait()
    if with_deg:
      d_h[_K - 1].wait()

    plsc.subcore_barrier()

    def copy_out(base, total):
      ch = _chunks(total)
      r_h, w_h = {}, {}
      r_h[0] = pltpu.async_copy(acc.at[pl.ds(base + ch[0][0], ch[0][1])],
                                rows[0].at[pl.ds(0, ch[0][1])], sem_g)
      for idx, (o, n) in enumerate(ch):
        b = idx % 2
        r_h[idx].wait()
        if idx >= 1:
          w_h[idx - 1].wait()
        if idx + 1 < len(ch):
          o2, n2 = ch[idx + 1]
          r_h[idx + 1] = pltpu.async_copy(acc.at[pl.ds(base + o2, n2)],
                                          rows[1 - b].at[pl.ds(0, n2)], sem_g)
        w_h[idx] = pltpu.async_copy(rows[b].at[pl.ds(0, n)],
                                    out_h.at[c, pl.ds(base + o, n)], sem_s)
      w_h[len(ch) - 1].wait()

    if with_deg:
      dr = pltpu.async_copy(dacc.at[pl.ds(s * _DPT, _DPT)], dstage, sem_i)
    copy_out(s * _OPT, _OPT)
    if with_deg:
      dr.wait()
      pltpu.sync_copy(dstage, deg_h.at[c, pl.ds(s * _DPT, _DPT)])

    @pl.when(s == _NS - 1)
    def _():
      copy_out(_OPT * _NS, _TAIL)

  return pl.kernel(body, out_type=out_type, mesh=mesh, scratch_types=scratch)


_seg_deg = _seg_builder(True)
_seg = _seg_builder(False)


def _dot_t(a, b):
  return lax.dot_general(a, b, (((1,), (1,)), ((), ())),
                         precision=lax.Precision.DEFAULT,
                         preferred_element_type=jnp.float32)


_BLK = 2000


def _dense0_body(x, w1r, xr):
  xr[...] = _dot_t(x[...], w1r[...])


def _dense0(x, W1r):
  return pl.pallas_call(
      _dense0_body,
      grid=(_N // _BLK,),
      in_specs=[
          pl.BlockSpec((_BLK, _D), lambda i: (i, 0)),
          pl.BlockSpec((_HID, _D), lambda i: (0, 0)),
      ],
      out_specs=pl.BlockSpec((_BLK, _HID), lambda i: (i, 0)),
      out_shape=jax.ShapeDtypeStruct((_N, _HID), jnp.float32),
  )(x, W1r)


def _dense1_body(s1p, degp, xr, w1l, b1l, w2l, w2r, p2, q2):
  deg = degp[0] + degp[1]
  rinv = 1.0 / jnp.maximum(deg, 1.0)
  agg = (s1p[0] + s1p[1]) * rinv
  h = _dot_t(agg, w1l[...]) + b1l[...] + xr[...]
  h = jnp.maximum(h, 0.0)
  p2[...] = _dot_t(h, w2l[...])
  q2[...] = _dot_t(h, w2r[...])


def _dense1(s1p, degp, xr, W1l, b1l, W2l, W2r):
  return pl.pallas_call(
      _dense1_body,
      grid=(_N // _BLK,),
      in_specs=[
          pl.BlockSpec((2, _BLK, _D), lambda i: (0, i, 0)),
          pl.BlockSpec((2, _BLK, 1), lambda i: (0, i, 0)),
          pl.BlockSpec((_BLK, _HID), lambda i: (i, 0)),
          pl.BlockSpec((_HID, _D), lambda i: (0, 0)),
          pl.BlockSpec((1, _HID), lambda i: (0, 0)),
          pl.BlockSpec((_D, _HID), lambda i: (0, 0)),
          pl.BlockSpec((_D, _HID), lambda i: (0, 0)),
      ],
      out_specs=[
          pl.BlockSpec((_BLK, _D), lambda i: (i, 0)),
          pl.BlockSpec((_BLK, _D), lambda i: (i, 0)),
      ],
      out_shape=[jax.ShapeDtypeStruct((_N, _D), jnp.float32)] * 2,
  )(s1p, degp, xr, W1l, b1l, W2l, W2r)


def _dense2_body(s2p, degp, q2, b2l, out):
  deg = degp[0] + degp[1]
  out[...] = (s2p[0] + s2p[1]) / jnp.maximum(deg, 1.0) + b2l[...] + q2[...]


def _dense2(s2p, degp, q2, b2l):
  return pl.pallas_call(
      _dense2_body,
      grid=(_N // _BLK,),
      in_specs=[
          pl.BlockSpec((2, _BLK, _D), lambda i: (0, i, 0)),
          pl.BlockSpec((2, _BLK, 1), lambda i: (0, i, 0)),
          pl.BlockSpec((_BLK, _D), lambda i: (i, 0)),
          pl.BlockSpec((1, _D), lambda i: (0, 0)),
      ],
      out_specs=pl.BlockSpec((_BLK, _D), lambda i: (i, 0)),
      out_shape=jax.ShapeDtypeStruct((_N, _D), jnp.float32),
  )(s2p, degp, q2, b2l)


def kernel(x, edge_index, W1l, b1l, W1r, W2l, b2l, W2r):
  pad = _EPAD - _E
  i = np.arange(pad)
  pad_e = jnp.asarray(np.stack([(i * 37) % _N,
                                _N + (i % (_ACC_ROWS - _N))]), dtype=jnp.int32)
  edge3 = jnp.concatenate([edge_index.astype(jnp.int32), pad_e],
                          axis=1).reshape(2, _NW, _K, _CHUNK)
  zr = jnp.zeros((_CHUNK, _D), jnp.float32)

  xr = _dense0(x, W1r)
  s1p, deg_flat = _seg_deg(x, edge3, zr)
  degp = deg_flat[:, :_N, None]
  p2, q2 = _dense1(s1p, degp, xr, W1l, b1l.reshape(1, -1), W2l, W2r)
  (s2p,) = _seg(p2, edge3, zr)
  return _dense2(s2p, degp, q2, b2l.reshape(1, -1))
